# baseline (speedup 1.0000x reference)
import jax
import jax.numpy as jnp
from jax.experimental import pallas as pl

_NUM_CLASSES = 1000


def _onehot_block(x_ref, o_ref):
    ids = jax.lax.broadcasted_iota(jnp.int32, o_ref.shape, 1)
    o_ref[...] = (ids == x_ref[...]).astype(jnp.float32)


def kernel(x):
    b, c = x.shape
    n = b * c
    xf = x.reshape(n, 1).astype(jnp.int32)
    rows = 512
    out = pl.pallas_call(
        _onehot_block,
        grid=(n // rows,),
        in_specs=[pl.BlockSpec((rows, 1), lambda i: (i, 0))],
        out_specs=pl.BlockSpec((rows, _NUM_CLASSES), lambda i: (i, 0)),
        out_shape=jax.ShapeDtypeStruct((n, _NUM_CLASSES), jnp.float32),
    )(xf)
    return out.reshape(b, c, _NUM_CLASSES)


# speedup vs baseline: 1.5676x; 1.5676x over previous
import jax
import jax.numpy as jnp
from jax.experimental import pallas as pl

_NUM_CLASSES = 1000
_ROWS = 32


def _onehot_block(x_ref, o_ref):
    ids = jax.lax.broadcasted_iota(jnp.int32, o_ref.shape, 2)
    o_ref[...] = (ids == x_ref[...][:, :, None]).astype(jnp.float32)


def kernel(x):
    b, c = x.shape
    return pl.pallas_call(
        _onehot_block,
        grid=(b // _ROWS,),
        in_specs=[pl.BlockSpec((_ROWS, c), lambda i: (i, 0))],
        out_specs=pl.BlockSpec((_ROWS, c, _NUM_CLASSES), lambda i: (i, 0, 0)),
        out_shape=jax.ShapeDtypeStruct((b, c, _NUM_CLASSES), jnp.float32),
    )(x.astype(jnp.int32))


# speedup vs baseline: 2.7022x; 1.7237x over previous
import jax
import jax.numpy as jnp
from jax.experimental import pallas as pl

_NUM_CLASSES = 1000
_CK = 8


def _onehot_block(x_ref, o_ref, *, ck):
    kb = pl.program_id(1)
    ids = jax.lax.broadcasted_iota(jnp.int32, o_ref.shape, 1) + kb * ck
    o_ref[...] = (ids == x_ref[...]).astype(jnp.float32)


def kernel(x):
    b, c = x.shape
    xt = x.T.astype(jnp.int32).reshape(c, 1, b)
    import functools
    out = pl.pallas_call(
        functools.partial(_onehot_block, ck=_CK),
        grid=(c, _NUM_CLASSES // _CK),
        in_specs=[pl.BlockSpec((1, 1, b), lambda j, kb: (j, 0, 0))],
        out_specs=pl.BlockSpec((1, _CK, b), lambda j, kb: (j, kb, 0)),
        out_shape=jax.ShapeDtypeStruct((c, _NUM_CLASSES, b), jnp.float32),
    )(xt)
    return jnp.transpose(out, (2, 0, 1))
